# baseline (speedup 1.0000x reference)
import functools

import jax
import jax.numpy as jnp
from jax import lax
from jax.experimental import pallas as pl
from jax.experimental.pallas import tpu as pltpu
from jax.experimental.pallas import tpu_sc as plsc

N = 10000
E = 320000
D = 128
H = 128
C = 40

DP = 144
DH = DP // 2
NC = 2
NS = 16
BLK = 128
NB = 160
EPAD = NS * NB * BLK
NA = N + 128
ZR = NA // NS
RPT = N // NS


def _sc_agg_kernel(table_hbm, src_hbm, dst_hbm, out_hbm,
                   src_v, dst_v, rows_v, rows2_v, rows3_v, rows4_v, agg_sh,
                   gsem, gsem2, gsem3, gsem4):
    c = lax.axis_index("c")
    s = lax.axis_index("s")

    zoffs = sorted({min(j * 16, DH - 16) for j in range(-(-DH // 16))})

    def zi(i, carry):
        for off in zoffs:
            rows_v[i, pl.ds(off, 16)] = jnp.zeros((16,), jnp.float32)
        return carry
    lax.fori_loop(0, BLK, zi, 0)

    for k in range(ZR // BLK):
        pltpu.sync_copy(rows_v, agg_sh.at[pl.ds(s * ZR + k * BLK, BLK)])
    ztail = ZR % BLK
    if ztail:
        pltpu.sync_copy(rows_v.at[pl.ds(0, ztail)],
                        agg_sh.at[pl.ds(s * ZR + ZR - ztail, ztail)])

    pltpu.sync_copy(src_hbm.at[c, s], src_v)
    pltpu.sync_copy(dst_hbm.at[s], dst_v)

    plsc.subcore_barrier()

    def start(b, buf, sem):
        pltpu.async_copy(table_hbm.at[src_v.at[b]], buf, sem)

    def drain(buf, sem):
        pltpu.make_async_copy(table_hbm.at[pl.ds(0, BLK)], buf, sem).wait()

    bufs = (rows_v, rows2_v, rows3_v, rows4_v)
    sems = (gsem, gsem2, gsem3, gsem4)
    nd = len(bufs)
    for k in range(nd):
        start(k, bufs[k], sems[k])

    def eb(g, carry):
        for k in range(nd):
            b = nd * g + k
            drain(bufs[k], sems[k])
            pltpu.sync_copy(bufs[k], agg_sh.at[dst_v.at[b]], add=True)
            start(lax.rem(b + nd, NB), bufs[k], sems[k])
        return carry
    lax.fori_loop(0, NB // nd, eb, 0)
    for k in range(nd):
        drain(bufs[k], sems[k])

    plsc.subcore_barrier()

    pltpu.sync_copy(agg_sh.at[pl.ds(s * RPT, RPT)],
                    out_hbm.at[c, pl.ds(s * RPT, RPT)])


_sc_agg = functools.partial(
    pl.kernel,
    out_type=jax.ShapeDtypeStruct((NC, N, DH), jnp.float32),
    mesh=plsc.VectorSubcoreMesh(core_axis_name="c", subcore_axis_name="s"),
    scratch_types=[
        pltpu.VMEM((NB, BLK), jnp.int32),
        pltpu.VMEM((NB, BLK), jnp.int32),
        pltpu.VMEM((BLK, DH), jnp.float32),
        pltpu.VMEM((BLK, DH), jnp.float32),
        pltpu.VMEM((BLK, DH), jnp.float32),
        pltpu.VMEM((BLK, DH), jnp.float32),
        pltpu.VMEM_SHARED((NA, DH), jnp.float32),
        pltpu.SemaphoreType.DMA,
        pltpu.SemaphoreType.DMA,
        pltpu.SemaphoreType.DMA,
        pltpu.SemaphoreType.DMA,
    ],
    compiler_params=pltpu.CompilerParams(use_tc_tiling_on_sc=False),
)(_sc_agg_kernel)


R = 2000


def _dense_block(hl, hr, al, ar, ws_ref, wn_ref, b_ref):
    h = jnp.concatenate([hl, hr], axis=1)[:, :D]
    a = jnp.concatenate([al, ar], axis=1)
    deg = jnp.maximum(a[:, D:D + 1], 1.0)
    agg = a[:, :D] / deg
    z = (jnp.dot(h, ws_ref[...], preferred_element_type=jnp.float32)
         + jnp.dot(agg, wn_ref[...], preferred_element_type=jnp.float32)
         + b_ref[...])
    return jnp.maximum(z, 0.0)


def _tc_layer_body(hl_ref, hr_ref, al_ref, ar_ref, ws_ref, wn_ref, b_ref,
                   out_ref):
    z = _dense_block(hl_ref[0], hr_ref[0], al_ref[0], ar_ref[0],
                     ws_ref, wn_ref, b_ref)
    pad = (lax.broadcasted_iota(jnp.int32, (R, DP - D), 1) == 0)
    zp = jnp.concatenate([z, pad.astype(jnp.float32)], axis=1)
    out_ref[...] = zp.reshape(R, 2, DH).swapaxes(0, 1)


def _tc_layer(table, agg, ws, wn, b):
    return pl.pallas_call(
        _tc_layer_body,
        grid=(N // R,),
        in_specs=[
            pl.BlockSpec((1, R, DH), lambda i: (0, i, 0)),
            pl.BlockSpec((1, R, DH), lambda i: (1, i, 0)),
            pl.BlockSpec((1, R, DH), lambda i: (0, i, 0)),
            pl.BlockSpec((1, R, DH), lambda i: (1, i, 0)),
            pl.BlockSpec((D, H), lambda i: (0, 0)),
            pl.BlockSpec((D, H), lambda i: (0, 0)),
            pl.BlockSpec((1, H), lambda i: (0, 0)),
        ],
        out_specs=pl.BlockSpec((2, R, DH), lambda i: (0, i, 0)),
        out_shape=jax.ShapeDtypeStruct((2, N, DH), jnp.float32),
    )(table, table, agg, agg, ws, wn, b.reshape(1, H))


def _tc_final_body(hl_ref, hr_ref, al_ref, ar_ref, ws_ref, wn_ref, b_ref,
                   wp_ref, bp_ref, score_ref, pls_ref, acc_ref):
    i = pl.program_id(0)
    z = _dense_block(hl_ref[0], hr_ref[0], al_ref[0], ar_ref[0],
                     ws_ref, wn_ref, b_ref)
    csum = jnp.sum(z, axis=0, keepdims=True)

    @pl.when(i == 0)
    def _():
        acc_ref[...] = csum

    @pl.when(i > 0)
    def _():
        acc_ref[...] = acc_ref[...] + csum

    @pl.when(i == pl.num_programs(0) - 1)
    def _():
        pls = acc_ref[...] * (1.0 / N)
        pls_ref[...] = pls
        score_ref[...] = (jnp.dot(pls, wp_ref[...],
                                  preferred_element_type=jnp.float32)
                          + bp_ref[...])


def _tc_final(table, agg, ws, wn, b, wp, bp):
    return pl.pallas_call(
        _tc_final_body,
        grid=(N // R,),
        in_specs=[
            pl.BlockSpec((1, R, DH), lambda i: (0, i, 0)),
            pl.BlockSpec((1, R, DH), lambda i: (1, i, 0)),
            pl.BlockSpec((1, R, DH), lambda i: (0, i, 0)),
            pl.BlockSpec((1, R, DH), lambda i: (1, i, 0)),
            pl.BlockSpec((D, H), lambda i: (0, 0)),
            pl.BlockSpec((D, H), lambda i: (0, 0)),
            pl.BlockSpec((1, H), lambda i: (0, 0)),
            pl.BlockSpec((H, C), lambda i: (0, 0)),
            pl.BlockSpec((1, C), lambda i: (0, 0)),
        ],
        out_specs=[
            pl.BlockSpec((1, C), lambda i: (0, 0)),
            pl.BlockSpec((1, H), lambda i: (0, 0)),
        ],
        out_shape=[
            jax.ShapeDtypeStruct((1, C), jnp.float32),
            jax.ShapeDtypeStruct((1, H), jnp.float32),
        ],
        scratch_shapes=[pltpu.VMEM((1, H), jnp.float32)],
    )(table, table, agg, agg, ws, wn, b.reshape(1, H), wp, bp.reshape(1, C))


def kernel(inputs, edge_index, W_self0, W_neigh0, b0,
           W_self1, W_neigh1, b1, W_pred, b_pred):
    pad = jnp.tile(
        (jnp.arange(DP - D, dtype=jnp.int32) == 0).astype(jnp.float32)[None],
        (N, 1))
    table0 = jnp.stack([inputs[:, :DH],
                        jnp.concatenate([inputs[:, DH:], pad], axis=1)])
    dummy = jnp.arange(EPAD - E, dtype=jnp.int32)
    src = jnp.concatenate(
        [edge_index[0], dummy % N]).reshape(NS, NB, BLK)
    src2 = jnp.stack([src, src + N])
    dst3 = jnp.concatenate(
        [edge_index[1], N + dummy % (NA - N)]).reshape(NS, NB, BLK)

    agg0 = _sc_agg(table0.reshape(NC * N, DH), src2, dst3)
    table1 = _tc_layer(table0, agg0, W_self0, W_neigh0, b0)
    agg1 = _sc_agg(table1.reshape(NC * N, DH), src2, dst3)
    score, pls = _tc_final(table1, agg1, W_self1, W_neigh1, b1,
                           W_pred, b_pred)
    return (score, pls)


# speedup vs baseline: 1.0552x; 1.0552x over previous
import functools

import jax
import jax.numpy as jnp
from jax import lax
from jax.experimental import pallas as pl
from jax.experimental.pallas import tpu as pltpu
from jax.experimental.pallas import tpu_sc as plsc

N = 10000
E = 320000
D = 128
H = 128
C = 40

DP = 144
DH = DP // 2
NC = 2
NS = 16
BLK = 128
NB = 160
EPAD = NS * NB * BLK
NA = N + 128
ZR = NA // NS
RPT = N // NS


def _make_sc_agg(dh):
    def body(table_hbm, src_hbm, dst_hbm, out_hbm,
             src_v, dst_v, rows_v, rows2_v, rows3_v, rows4_v, agg_sh,
             gsem, gsem2, gsem3, gsem4):
        c = lax.axis_index("c")
        s = lax.axis_index("s")

        zoffs = sorted({min(j * 16, dh - 16) for j in range(-(-dh // 16))})

        def zi(i, carry):
            for off in zoffs:
                rows_v[i, pl.ds(off, 16)] = jnp.zeros((16,), jnp.float32)
            return carry
        lax.fori_loop(0, BLK, zi, 0)

        for k in range(ZR // BLK):
            pltpu.sync_copy(rows_v, agg_sh.at[pl.ds(s * ZR + k * BLK, BLK)])
        ztail = ZR % BLK
        if ztail:
            pltpu.sync_copy(rows_v.at[pl.ds(0, ztail)],
                            agg_sh.at[pl.ds(s * ZR + ZR - ztail, ztail)])

        pltpu.sync_copy(src_hbm.at[c, s], src_v)
        pltpu.sync_copy(dst_hbm.at[s], dst_v)

        plsc.subcore_barrier()

        def start(b, buf, sem):
            pltpu.async_copy(table_hbm.at[src_v.at[b]], buf, sem)

        def drain(buf, sem):
            pltpu.make_async_copy(table_hbm.at[pl.ds(0, BLK)], buf,
                                  sem).wait()

        bufs = (rows_v, rows2_v, rows3_v, rows4_v)
        sems = (gsem, gsem2, gsem3, gsem4)
        nd = len(bufs)
        for k in range(nd):
            start(k, bufs[k], sems[k])

        def eb(g, carry):
            for k in range(nd):
                b = nd * g + k
                drain(bufs[k], sems[k])
                pltpu.sync_copy(bufs[k], agg_sh.at[dst_v.at[b]], add=True)
                start(lax.rem(b + nd, NB), bufs[k], sems[k])
            return carry
        lax.fori_loop(0, NB // nd, eb, 0)
        for k in range(nd):
            drain(bufs[k], sems[k])

        plsc.subcore_barrier()

        pltpu.sync_copy(agg_sh.at[pl.ds(s * RPT, RPT)],
                        out_hbm.at[c, pl.ds(s * RPT, RPT)])

    return functools.partial(
        pl.kernel,
        out_type=jax.ShapeDtypeStruct((NC, N, dh), jnp.float32),
        mesh=plsc.VectorSubcoreMesh(core_axis_name="c",
                                    subcore_axis_name="s"),
        scratch_types=[
            pltpu.VMEM((NB, BLK), jnp.int32),
            pltpu.VMEM((NB, BLK), jnp.int32),
            pltpu.VMEM((BLK, dh), jnp.float32),
            pltpu.VMEM((BLK, dh), jnp.float32),
            pltpu.VMEM((BLK, dh), jnp.float32),
            pltpu.VMEM((BLK, dh), jnp.float32),
            pltpu.VMEM_SHARED((NA, dh), jnp.float32),
            pltpu.SemaphoreType.DMA,
            pltpu.SemaphoreType.DMA,
            pltpu.SemaphoreType.DMA,
            pltpu.SemaphoreType.DMA,
        ],
        compiler_params=pltpu.CompilerParams(use_tc_tiling_on_sc=False),
    )(body)


_sc_agg = _make_sc_agg(DH)
DH1 = D // 2
_sc_agg1 = _make_sc_agg(DH1)


R = 2000


def _dense_block(hl, hr, al, ar, ws_ref, wn_ref, b_ref):
    h = jnp.concatenate([hl, hr], axis=1)[:, :D]
    a = jnp.concatenate([al, ar], axis=1)
    deg = jnp.maximum(a[:, D:D + 1], 1.0)
    agg = a[:, :D] / deg
    z = (jnp.dot(h, ws_ref[...], preferred_element_type=jnp.float32)
         + jnp.dot(agg, wn_ref[...], preferred_element_type=jnp.float32)
         + b_ref[...])
    return jnp.maximum(z, 0.0)


def _tc_layer_body(hl_ref, hr_ref, al_ref, ar_ref, ws_ref, wn_ref, b_ref,
                   out_ref):
    z = _dense_block(hl_ref[0], hr_ref[0], al_ref[0], ar_ref[0],
                     ws_ref, wn_ref, b_ref)
    out_ref[...] = z.reshape(R, 2, DH1).swapaxes(0, 1)


def _tc_layer(table, agg, ws, wn, b):
    return pl.pallas_call(
        _tc_layer_body,
        grid=(N // R,),
        in_specs=[
            pl.BlockSpec((1, R, DH), lambda i: (0, i, 0)),
            pl.BlockSpec((1, R, DH), lambda i: (1, i, 0)),
            pl.BlockSpec((1, R, DH), lambda i: (0, i, 0)),
            pl.BlockSpec((1, R, DH), lambda i: (1, i, 0)),
            pl.BlockSpec((D, H), lambda i: (0, 0)),
            pl.BlockSpec((D, H), lambda i: (0, 0)),
            pl.BlockSpec((1, H), lambda i: (0, 0)),
        ],
        out_specs=pl.BlockSpec((2, R, DH1), lambda i: (0, i, 0)),
        out_shape=jax.ShapeDtypeStruct((2, N, DH1), jnp.float32),
    )(table, table, agg, agg, ws, wn, b.reshape(1, H))


def _tc_final_body(hl_ref, hr_ref, al_ref, ar_ref, adeg_ref,
                   ws_ref, wn_ref, b_ref,
                   wp_ref, bp_ref, score_ref, pls_ref, acc_ref):
    i = pl.program_id(0)
    h = jnp.concatenate([hl_ref[0], hr_ref[0]], axis=1)
    a = jnp.concatenate([al_ref[0], ar_ref[0]], axis=1)
    deg = jnp.maximum(adeg_ref[0][:, D - DH:D - DH + 1], 1.0)
    agg = a / deg
    z = (jnp.dot(h, ws_ref[...], preferred_element_type=jnp.float32)
         + jnp.dot(agg, wn_ref[...], preferred_element_type=jnp.float32)
         + b_ref[...])
    z = jnp.maximum(z, 0.0)
    csum = jnp.sum(z, axis=0, keepdims=True)

    @pl.when(i == 0)
    def _():
        acc_ref[...] = csum

    @pl.when(i > 0)
    def _():
        acc_ref[...] = acc_ref[...] + csum

    @pl.when(i == pl.num_programs(0) - 1)
    def _():
        pls = acc_ref[...] * (1.0 / N)
        pls_ref[...] = pls
        score_ref[...] = (jnp.dot(pls, wp_ref[...],
                                  preferred_element_type=jnp.float32)
                          + bp_ref[...])


def _tc_final(table, agg, agg0, ws, wn, b, wp, bp):
    return pl.pallas_call(
        _tc_final_body,
        grid=(N // R,),
        in_specs=[
            pl.BlockSpec((1, R, DH1), lambda i: (0, i, 0)),
            pl.BlockSpec((1, R, DH1), lambda i: (1, i, 0)),
            pl.BlockSpec((1, R, DH1), lambda i: (0, i, 0)),
            pl.BlockSpec((1, R, DH1), lambda i: (1, i, 0)),
            pl.BlockSpec((1, R, DH), lambda i: (1, i, 0)),
            pl.BlockSpec((D, H), lambda i: (0, 0)),
            pl.BlockSpec((D, H), lambda i: (0, 0)),
            pl.BlockSpec((1, H), lambda i: (0, 0)),
            pl.BlockSpec((H, C), lambda i: (0, 0)),
            pl.BlockSpec((1, C), lambda i: (0, 0)),
        ],
        out_specs=[
            pl.BlockSpec((1, C), lambda i: (0, 0)),
            pl.BlockSpec((1, H), lambda i: (0, 0)),
        ],
        out_shape=[
            jax.ShapeDtypeStruct((1, C), jnp.float32),
            jax.ShapeDtypeStruct((1, H), jnp.float32),
        ],
        scratch_shapes=[pltpu.VMEM((1, H), jnp.float32)],
    )(table, table, agg, agg, agg0, ws, wn, b.reshape(1, H), wp,
      bp.reshape(1, C))


def kernel(inputs, edge_index, W_self0, W_neigh0, b0,
           W_self1, W_neigh1, b1, W_pred, b_pred):
    pad = jnp.tile(
        (jnp.arange(DP - D, dtype=jnp.int32) == 0).astype(jnp.float32)[None],
        (N, 1))
    table0 = jnp.stack([inputs[:, :DH],
                        jnp.concatenate([inputs[:, DH:], pad], axis=1)])
    dummy = jnp.arange(EPAD - E, dtype=jnp.int32)
    src = jnp.concatenate(
        [edge_index[0], dummy % N]).reshape(NS, NB, BLK)
    src2 = jnp.stack([src, src + N])
    dst3 = jnp.concatenate(
        [edge_index[1], N + dummy % (NA - N)]).reshape(NS, NB, BLK)

    agg0 = _sc_agg(table0.reshape(NC * N, DH), src2, dst3)
    table1 = _tc_layer(table0, agg0, W_self0, W_neigh0, b0)
    agg1 = _sc_agg1(table1.reshape(NC * N, DH1), src2, dst3)
    score, pls = _tc_final(table1, agg1, agg0, W_self1, W_neigh1, b1,
                           W_pred, b_pred)
    return (score, pls)
